# baseline (speedup 1.0000x reference)
import functools

import jax
import jax.numpy as jnp
from jax import lax
from jax.experimental import pallas as pl
from jax.experimental.pallas import tpu as pltpu
from jax.experimental.pallas import tpu_sc as plsc

HIDDEN = 128
HEADS = 8
HD = 16
NN = 10000
NE = 320000
NPAD = 10240
WQ = 128
WK = 272
VOFF = 144
WA = 136
SCALE = HD ** -0.5

NC, NS = 2, 16
B = 32
NB = 316
EPAD = NC * NS * NB * B
RPS = NPAD // NS

R = 512


def _pre_body(x_ref, wq, bq, wk, bk, wv, bv, we, weap,
              qa_ref, kv_ref):
    x = x_ref[...]
    wc = jnp.dot(we[...], weap[...], preferred_element_type=jnp.float32)
    qa_ref[...] = jnp.dot(x, wq[...], preferred_element_type=jnp.float32) + bq[...]
    kv_ref[:, :HIDDEN] = jnp.dot(x, wk[...], preferred_element_type=jnp.float32) + bk[...]
    kv_ref[:, HIDDEN:VOFF] = jnp.dot(x, wc[:HIDDEN], preferred_element_type=jnp.float32)
    kv_ref[:, VOFF:] = jnp.dot(x, wv[...], preferred_element_type=jnp.float32) + bv[...]


def _edge_body(qa_hbm, kv_hbm, src_hbm, tgt_hbm, out_hbm,
               s0, s1, s2, s3, t0, t1, t2, t3,
               qa0, qa1, kv0, kv1, cb0, cb1,
               ise0, ise1, ise2, ise3, gse0, gse1, sse0, sse1,
               acc_sh):
    srcs = (s0, s1, s2, s3)
    tgts = (t0, t1, t2, t3)
    qas = (qa0, qa1)
    kvs = (kv0, kv1)
    cbs = (cb0, cb1)
    isems = (ise0, ise1, ise2, ise3)
    gsems = (gse0, gse1)
    ssems = (sse0, sse1)

    cid = lax.axis_index("c")
    sid = lax.axis_index("s")
    wid = sid * NC + cid

    zv = jnp.zeros((16,), jnp.float32)

    def zrow(r, _):
        for cc in range(WA // 16):
            cb0[r, pl.ds(cc * 16, 16)] = zv
        cb0[r, pl.ds(WA - 16, 16)] = zv
        return 0
    lax.fori_loop(0, B, zrow, 0)

    def zchunk(i, _):
        pltpu.sync_copy(cb0, acc_sh.at[pl.ds(sid * RPS + i * B, B)])
        return 0
    lax.fori_loop(0, RPS // B, zchunk, 0)
    plsc.subcore_barrier()

    lanes = lax.iota(jnp.int32, 16)

    def fire_idx(q, b):
        pltpu.async_copy(src_hbm.at[pl.ds((wid * NB + b) * B, B)], srcs[q], isems[q])
        pltpu.async_copy(tgt_hbm.at[pl.ds((wid * NB + b) * B, B)], tgts[q], isems[q])

    def wait_idx(q):
        pltpu.make_async_copy(src_hbm.at[pl.ds(0, B)], srcs[q], isems[q]).wait()
        pltpu.make_async_copy(tgt_hbm.at[pl.ds(0, B)], tgts[q], isems[q]).wait()

    def fire_gathers(p, q):
        pltpu.async_copy(qa_hbm.at[tgts[q]], qas[p], gsems[p])
        pltpu.async_copy(kv_hbm.at[srcs[q]], kvs[p], gsems[p])

    def wait_gathers(p):
        pltpu.make_async_copy(qa_hbm.at[tgts[0]], qas[p], gsems[p]).wait()
        pltpu.make_async_copy(kv_hbm.at[srcs[0]], kvs[p], gsems[p]).wait()

    def wait_scatter(p):
        pltpu.make_async_copy(cbs[p], acc_sh.at[tgts[0]], ssems[p]).wait()

    def compute(p):
        qa_r, kv_r, comb = qas[p], kvs[p], cbs[p]

        def group(g, _):
            rows = lanes + g * 16
            for h in range(HEADS):
                cb = jnp.full((16,), HIDDEN + h, jnp.int32)
                bias = plsc.load_gather(kv_r, [rows, cb])
                acc = jnp.zeros((16,), jnp.float32)
                for d in range(HD):
                    cc = jnp.full((16,), h * HD + d, jnp.int32)
                    acc = acc + (plsc.load_gather(qa_r, [rows, cc])
                                 * plsc.load_gather(kv_r, [rows, cc]))
                s = acc * SCALE + bias
                t = jnp.exp(jnp.clip(s, -60.0, 60.0))
                plsc.store_scatter(comb, [rows, cb], t)
                for d in range(HD):
                    cv = jnp.full((16,), VOFF + h * HD + d, jnp.int32)
                    cc = jnp.full((16,), h * HD + d, jnp.int32)
                    vv = plsc.load_gather(kv_r, [rows, cv])
                    plsc.store_scatter(comb, [rows, cc], vv * t)
            return 0
        lax.fori_loop(0, B // 16, group, 0)

    fire_idx(0, 0)
    fire_idx(1, 1)
    wait_idx(0)
    fire_gathers(0, 0)

    def body(j, _):
        for ph in range(4):
            p = ph % 2
            b = j * 4 + ph

            if ph >= 2:
                wait_scatter(p)
            else:
                @pl.when(j > 0)
                def _():
                    wait_scatter(p)

            @pl.when(b + 2 < NB)
            def _():
                fire_idx((ph + 2) % 4, b + 2)

            @pl.when(b + 1 < NB)
            def _():
                wait_idx((ph + 1) % 4)
                fire_gathers((ph + 1) % 2, (ph + 1) % 4)

            wait_gathers(p)
            compute(p)
            pltpu.async_copy(cbs[p], acc_sh.at[tgts[ph]], ssems[p],
                             add=True)
        return 0
    lax.fori_loop(0, NB // 4, body, 0)
    wait_scatter(0)
    wait_scatter(1)

    plsc.subcore_barrier()

    def drain(i, _):
        r0 = sid * RPS + i * B
        pltpu.sync_copy(acc_sh.at[pl.ds(r0, B)], cb0)
        pltpu.sync_copy(cb0, out_hbm.at[cid, pl.ds(r0, B)])
        return 0
    lax.fori_loop(0, RPS // B, drain, 0)


_edge_call = functools.partial(
    pl.kernel,
    out_type=jax.ShapeDtypeStruct((NC, NPAD, WA), jnp.float32),
    mesh=plsc.VectorSubcoreMesh(core_axis_name="c", subcore_axis_name="s"),
    compiler_params=pltpu.CompilerParams(use_tc_tiling_on_sc=False,
                                         needs_layout_passes=False),
    scratch_types=(
        [pltpu.VMEM((B,), jnp.int32)] * 8
        + [pltpu.VMEM((B, WQ), jnp.float32)] * 2
        + [pltpu.VMEM((B, WK), jnp.float32)] * 2
        + [pltpu.VMEM((B, WA), jnp.float32)] * 2
        + [pltpu.SemaphoreType.DMA] * 8
        + [pltpu.VMEM_SHARED((NPAD, WA), jnp.float32)]
    ),
)(_edge_body)


def _post_body(acc_ref, wo, bo, o_ref):
    a = acc_ref[0] + acc_ref[1]
    num = a[:, :HIDDEN]
    den = a[:, HIDDEN:HIDDEN + HEADS]
    recip = jnp.where(den > 0, 1.0 / den, 0.0)
    i0 = lax.broadcasted_iota(jnp.int32, (HEADS, HIDDEN), 0)
    i1 = lax.broadcasted_iota(jnp.int32, (HEADS, HIDDEN), 1)
    sel = (i1 // HD == i0).astype(jnp.float32)
    den128 = jnp.dot(recip, sel, preferred_element_type=jnp.float32)
    o_ref[...] = (jnp.dot(num * den128, wo[...],
                          preferred_element_type=jnp.float32) + bo[...])


def kernel(x, edge_index, Wq, bq, Wk, bk, Wv, bv, We, be, Wea, bea, Wo, bo):
    x2d = jnp.pad(x[0], ((0, NPAD - NN), (0, 0)))
    pad = jnp.full((EPAD - NE,), NPAD - 1, jnp.int32)
    src = jnp.concatenate([edge_index[0, :, 0], pad])
    tgt = jnp.concatenate([edge_index[0, :, 1], pad])
    weap = jnp.pad(Wea, ((0, 0), (0, HD - HEADS)))

    full = lambda s: pl.BlockSpec(s, lambda i: (0,) * len(s))
    qa, kv = pl.pallas_call(
        _pre_body,
        grid=(NPAD // R,),
        in_specs=[
            pl.BlockSpec((R, HIDDEN), lambda i: (i, 0)),
            full((HIDDEN, HIDDEN)), full((1, HIDDEN)),
            full((HIDDEN, HIDDEN)), full((1, HIDDEN)),
            full((HIDDEN, HIDDEN)), full((1, HIDDEN)),
            full((2 * HIDDEN, 64)), full((64, HD)),
        ],
        out_specs=[
            pl.BlockSpec((R, WQ), lambda i: (i, 0)),
            pl.BlockSpec((R, WK), lambda i: (i, 0)),
        ],
        out_shape=[
            jax.ShapeDtypeStruct((NPAD, WQ), jnp.float32),
            jax.ShapeDtypeStruct((NPAD, WK), jnp.float32),
        ],
    )(x2d, Wq, bq.reshape(1, -1), Wk, bk.reshape(1, -1),
      Wv, bv.reshape(1, -1), We, weap)

    acc = _edge_call(qa, kv, src, tgt)

    out = pl.pallas_call(
        _post_body,
        grid=(NN // 400,),
        in_specs=[
            pl.BlockSpec((NC, 400, WA), lambda i: (0, i, 0)),
            full((HIDDEN, HIDDEN)), full((1, HIDDEN)),
        ],
        out_specs=pl.BlockSpec((400, HIDDEN), lambda i: (i, 0)),
        out_shape=jax.ShapeDtypeStruct((NN, HIDDEN), jnp.float32),
    )(acc, Wo, bo.reshape(1, -1))
    return out.reshape(1, NN, HIDDEN)


# speedup vs baseline: 1.2313x; 1.2313x over previous
import functools

import jax
import jax.numpy as jnp
from jax import lax
from jax.experimental import pallas as pl
from jax.experimental.pallas import tpu as pltpu
from jax.experimental.pallas import tpu_sc as plsc

HIDDEN = 128
HEADS = 8
HD = 16
NN = 10000
NE = 320000
NPAD = 10240
WQ = 144
WK = 272
VOFF = 144
WA = 136
SCALE = HD ** -0.5

NC, NS = 2, 16
B = 32
NB = 316
EPAD = NC * NS * NB * B
RPS = NPAD // NS

R = 512


def _pre_body(x_ref, wq, bq, wk, bk, wv, bv, we, weap,
              qa_ref, kv_ref):
    x = x_ref[...]
    wc = jnp.dot(we[...], weap[...], preferred_element_type=jnp.float32)
    qa_ref[:, :HIDDEN] = jnp.dot(x, wq[...], preferred_element_type=jnp.float32) + bq[...]
    qa_ref[:, HIDDEN:] = jnp.zeros((x.shape[0], WQ - HIDDEN), jnp.float32)
    kv_ref[:, :HIDDEN] = jnp.dot(x, wk[...], preferred_element_type=jnp.float32) + bk[...]
    kv_ref[:, HIDDEN:VOFF] = jnp.dot(x, wc[:HIDDEN], preferred_element_type=jnp.float32)
    kv_ref[:, VOFF:] = jnp.dot(x, wv[...], preferred_element_type=jnp.float32) + bv[...]


def _edge_body(qa_hbm, kv_hbm, src_hbm, tgt_hbm, out_hbm,
               s0, s1, s2, s3, t0, t1, t2, t3,
               qa0, qa1, kv0, kv1, cb0, cb1,
               ise0, ise1, ise2, ise3, gse0, gse1, sse0, sse1,
               acc_sh):
    srcs = (s0, s1, s2, s3)
    tgts = (t0, t1, t2, t3)
    qas = (qa0, qa1)
    kvs = (kv0, kv1)
    cbs = (cb0, cb1)
    isems = (ise0, ise1, ise2, ise3)
    gsems = (gse0, gse1)
    ssems = (sse0, sse1)

    cid = lax.axis_index("c")
    sid = lax.axis_index("s")
    wid = sid * NC + cid

    zv = jnp.zeros((16,), jnp.float32)

    def zrow(r, _):
        for cc in range(WA // 16):
            cb0[r, pl.ds(cc * 16, 16)] = zv
        cb0[r, pl.ds(WA - 16, 16)] = zv
        return 0
    lax.fori_loop(0, B, zrow, 0)

    def zchunk(i, _):
        pltpu.sync_copy(cb0, acc_sh.at[pl.ds(sid * RPS + i * B, B)])
        return 0
    lax.fori_loop(0, RPS // B, zchunk, 0)
    plsc.subcore_barrier()

    lanes = lax.iota(jnp.int32, 16)

    def fire_idx(q, b):
        pltpu.async_copy(src_hbm.at[pl.ds((wid * NB + b) * B, B)], srcs[q], isems[q])
        pltpu.async_copy(tgt_hbm.at[pl.ds((wid * NB + b) * B, B)], tgts[q], isems[q])

    def wait_idx(q):
        pltpu.make_async_copy(src_hbm.at[pl.ds(0, B)], srcs[q], isems[q]).wait()
        pltpu.make_async_copy(tgt_hbm.at[pl.ds(0, B)], tgts[q], isems[q]).wait()

    def fire_gathers(p, q):
        pltpu.async_copy(qa_hbm.at[tgts[q]], qas[p], gsems[p])
        pltpu.async_copy(kv_hbm.at[srcs[q]], kvs[p], gsems[p])

    def wait_gathers(p):
        pltpu.make_async_copy(qa_hbm.at[tgts[0]], qas[p], gsems[p]).wait()
        pltpu.make_async_copy(kv_hbm.at[srcs[0]], kvs[p], gsems[p]).wait()

    def wait_scatter(p):
        pltpu.make_async_copy(cbs[p], acc_sh.at[tgts[0]], ssems[p]).wait()

    def compute(p):
        qa_r, kv_r, comb = qas[p], kvs[p], cbs[p]

        def group(g, _):
            rows = lanes + g * 16
            for h in range(HEADS):
                cb = jnp.full((16,), HIDDEN + h, jnp.int32)
                bias = plsc.load_gather(kv_r, [rows, cb])
                acc = jnp.zeros((16,), jnp.float32)
                for d in range(HD):
                    cc = jnp.full((16,), h * HD + d, jnp.int32)
                    acc = acc + (plsc.load_gather(qa_r, [rows, cc])
                                 * plsc.load_gather(kv_r, [rows, cc]))
                s = acc * SCALE + bias
                t = jnp.exp(jnp.clip(s, -60.0, 60.0))
                plsc.store_scatter(comb, [rows, cb], t)
                for d in range(HD):
                    cv = jnp.full((16,), VOFF + h * HD + d, jnp.int32)
                    cc = jnp.full((16,), h * HD + d, jnp.int32)
                    vv = plsc.load_gather(kv_r, [rows, cv])
                    plsc.store_scatter(comb, [rows, cc], vv * t)
            return 0
        lax.fori_loop(0, B // 16, group, 0)

    fire_idx(0, 0)
    fire_idx(1, 1)
    wait_idx(0)
    fire_gathers(0, 0)

    def body(j, _):
        for ph in range(4):
            p = ph % 2
            b = j * 4 + ph

            if ph >= 2:
                wait_scatter(p)
            else:
                @pl.when(j > 0)
                def _():
                    wait_scatter(p)

            @pl.when(b + 2 < NB)
            def _():
                fire_idx((ph + 2) % 4, b + 2)

            @pl.when(b + 1 < NB)
            def _():
                wait_idx((ph + 1) % 4)
                fire_gathers((ph + 1) % 2, (ph + 1) % 4)

            wait_gathers(p)
            compute(p)
            pltpu.async_copy(cbs[p], acc_sh.at[tgts[ph]], ssems[p],
                             add=True)
        return 0
    lax.fori_loop(0, NB // 4, body, 0)
    wait_scatter(0)
    wait_scatter(1)

    plsc.subcore_barrier()

    def drain(i, _):
        r0 = sid * RPS + i * B
        pltpu.sync_copy(acc_sh.at[pl.ds(r0, B)], cb0)
        pltpu.sync_copy(cb0, out_hbm.at[cid, pl.ds(r0, B)])
        return 0
    lax.fori_loop(0, RPS // B, drain, 0)


_edge_call = functools.partial(
    pl.kernel,
    out_type=jax.ShapeDtypeStruct((NC, NPAD, WA), jnp.float32),
    mesh=plsc.VectorSubcoreMesh(core_axis_name="c", subcore_axis_name="s"),
    compiler_params=pltpu.CompilerParams(use_tc_tiling_on_sc=False,
                                         needs_layout_passes=False),
    scratch_types=(
        [pltpu.VMEM((B,), jnp.int32)] * 8
        + [pltpu.VMEM((B, WQ), jnp.float32)] * 2
        + [pltpu.VMEM((B, WK), jnp.float32)] * 2
        + [pltpu.VMEM((B, WA), jnp.float32)] * 2
        + [pltpu.SemaphoreType.DMA] * 8
        + [pltpu.VMEM_SHARED((NPAD, WA), jnp.float32)]
    ),
)(_edge_body)


def _post_body(acc_ref, wo, bo, o_ref):
    a = acc_ref[0] + acc_ref[1]
    num = a[:, :HIDDEN]
    den = a[:, HIDDEN:HIDDEN + HEADS]
    recip = jnp.where(den > 0, 1.0 / den, 0.0)
    i0 = lax.broadcasted_iota(jnp.int32, (HEADS, HIDDEN), 0)
    i1 = lax.broadcasted_iota(jnp.int32, (HEADS, HIDDEN), 1)
    sel = (i1 // HD == i0).astype(jnp.float32)
    den128 = jnp.dot(recip, sel, preferred_element_type=jnp.float32)
    o_ref[...] = (jnp.dot(num * den128, wo[...],
                          preferred_element_type=jnp.float32) + bo[...])


def kernel(x, edge_index, Wq, bq, Wk, bk, Wv, bv, We, be, Wea, bea, Wo, bo):
    x2d = jnp.pad(x[0], ((0, NPAD - NN), (0, 0)))
    pad = jnp.full((EPAD - NE,), NPAD - 1, jnp.int32)
    src = jnp.concatenate([edge_index[0, :, 0], pad])
    tgt = jnp.concatenate([edge_index[0, :, 1], pad])
    weap = jnp.pad(Wea, ((0, 0), (0, HD - HEADS)))

    full = lambda s: pl.BlockSpec(s, lambda i: (0,) * len(s))
    qa, kv = pl.pallas_call(
        _pre_body,
        grid=(NPAD // R,),
        in_specs=[
            pl.BlockSpec((R, HIDDEN), lambda i: (i, 0)),
            full((HIDDEN, HIDDEN)), full((1, HIDDEN)),
            full((HIDDEN, HIDDEN)), full((1, HIDDEN)),
            full((HIDDEN, HIDDEN)), full((1, HIDDEN)),
            full((2 * HIDDEN, 64)), full((64, HD)),
        ],
        out_specs=[
            pl.BlockSpec((R, WQ), lambda i: (i, 0)),
            pl.BlockSpec((R, WK), lambda i: (i, 0)),
        ],
        out_shape=[
            jax.ShapeDtypeStruct((NPAD, WQ), jnp.float32),
            jax.ShapeDtypeStruct((NPAD, WK), jnp.float32),
        ],
    )(x2d, Wq, bq.reshape(1, -1), Wk, bk.reshape(1, -1),
      Wv, bv.reshape(1, -1), We, weap)

    acc = _edge_call(qa, kv, src, tgt)

    out = pl.pallas_call(
        _post_body,
        grid=(NN // 400,),
        in_specs=[
            pl.BlockSpec((NC, 400, WA), lambda i: (0, i, 0)),
            full((HIDDEN, HIDDEN)), full((1, HIDDEN)),
        ],
        out_specs=pl.BlockSpec((400, HIDDEN), lambda i: (i, 0)),
        out_shape=jax.ShapeDtypeStruct((NN, HIDDEN), jnp.float32),
    )(acc, Wo, bo.reshape(1, -1))
    return out.reshape(1, NN, HIDDEN)


# speedup vs baseline: 1.7816x; 1.4469x over previous
import functools

import jax
import jax.numpy as jnp
from jax import lax
from jax.experimental import pallas as pl
from jax.experimental.pallas import tpu as pltpu
from jax.experimental.pallas import tpu_sc as plsc

HIDDEN = 128
HEADS = 8
HD = 16
NN = 10000
NE = 320000
NPAD = 10240
WQ = 144
WK = 144
AOFF = 64
VOFF = 80
WA = 136
SCALE = HD ** -0.5

NC, NS = 2, 16
B = 32
NB = 316
EPAD = NC * NS * NB * B
RPS = NPAD // NS

R = 512


def _rne16(f):
    bi = lax.bitcast_convert_type(f, jnp.int32)
    return ((bi + 0x7FFF + ((bi >> 16) & 1)) >> 16) & 0xFFFF


def _pre_body(x_ref, wq, bq, wk, bk, wv, bv, we, weap,
              qa_ref, kv_ref):
    x = x_ref[...]
    wc = jnp.dot(we[...], weap[...], preferred_element_type=jnp.float32)
    qa_ref[:, :HIDDEN] = jnp.dot(x, wq[...], preferred_element_type=jnp.float32) + bq[...]
    qa_ref[:, HIDDEN:] = jnp.zeros((x.shape[0], WQ - HIDDEN), jnp.float32)
    k = (jnp.dot(x, wk[...], preferred_element_type=jnp.float32) + bk[...])
    v = (jnp.dot(x, wv[...], preferred_element_type=jnp.float32) + bv[...])
    a = jnp.dot(x, wc[:HIDDEN], preferred_element_type=jnp.float32)[:, :HEADS]
    kv_ref[:, :AOFF] = _rne16(k[:, :AOFF]) | (_rne16(k[:, AOFF:]) << 16)
    kv_ref[:, AOFF:AOFF + HEADS] = lax.bitcast_convert_type(a, jnp.int32)
    kv_ref[:, AOFF + HEADS:VOFF] = jnp.zeros((x.shape[0], VOFF - AOFF - HEADS), jnp.int32)
    kv_ref[:, VOFF:] = _rne16(v[:, :AOFF]) | (_rne16(v[:, AOFF:]) << 16)


def _edge_body(qa_hbm, kv_hbm, src_hbm, tgt_hbm, out_hbm,
               s0, s1, s2, s3, t0, t1, t2, t3,
               qa0, qa1, kv0, kv1, cb0, cb1,
               ise0, ise1, ise2, ise3, gse0, gse1, sse0, sse1,
               acc_sh):
    srcs = (s0, s1, s2, s3)
    tgts = (t0, t1, t2, t3)
    qas = (qa0, qa1)
    kvs = (kv0, kv1)
    cbs = (cb0, cb1)
    isems = (ise0, ise1, ise2, ise3)
    gsems = (gse0, gse1)
    ssems = (sse0, sse1)

    cid = lax.axis_index("c")
    sid = lax.axis_index("s")
    wid = sid * NC + cid

    zv = jnp.zeros((16,), jnp.float32)

    def zrow(r, _):
        for cc in range(WA // 16):
            cb0[r, pl.ds(cc * 16, 16)] = zv
        cb0[r, pl.ds(WA - 16, 16)] = zv
        return 0
    lax.fori_loop(0, B, zrow, 0)

    def zchunk(i, _):
        pltpu.sync_copy(cb0, acc_sh.at[pl.ds(sid * RPS + i * B, B)])
        return 0
    lax.fori_loop(0, RPS // B, zchunk, 0)
    plsc.subcore_barrier()

    lanes = lax.iota(jnp.int32, 16)

    def fire_idx(q, b):
        pltpu.async_copy(src_hbm.at[pl.ds((wid * NB + b) * B, B)], srcs[q], isems[q])
        pltpu.async_copy(tgt_hbm.at[pl.ds((wid * NB + b) * B, B)], tgts[q], isems[q])

    def wait_idx(q):
        pltpu.make_async_copy(src_hbm.at[pl.ds(0, B)], srcs[q], isems[q]).wait()
        pltpu.make_async_copy(tgt_hbm.at[pl.ds(0, B)], tgts[q], isems[q]).wait()

    def fire_gathers(p, q):
        pltpu.async_copy(qa_hbm.at[tgts[q]], qas[p], gsems[p])
        pltpu.async_copy(kv_hbm.at[srcs[q]], kvs[p], gsems[p])

    def wait_gathers(p):
        pltpu.make_async_copy(qa_hbm.at[tgts[0]], qas[p], gsems[p]).wait()
        pltpu.make_async_copy(kv_hbm.at[srcs[0]], kvs[p], gsems[p]).wait()

    def wait_scatter(p):
        pltpu.make_async_copy(cbs[p], acc_sh.at[tgts[0]], ssems[p]).wait()

    def compute(p):
        qa_r, kv_r, comb = qas[p], kvs[p], cbs[p]

        def group(g, _):
            rows = lanes + g * 16
            for hp in range(HEADS // 2):
                blo = jnp.full((16,), AOFF + hp, jnp.int32)
                bhi = jnp.full((16,), AOFF + hp + 4, jnp.int32)
                bias_lo = plsc.bitcast(plsc.load_gather(kv_r, [rows, blo]),
                                       jnp.float32)
                bias_hi = plsc.bitcast(plsc.load_gather(kv_r, [rows, bhi]),
                                       jnp.float32)
                acc_lo = jnp.zeros((16,), jnp.float32)
                acc_hi = jnp.zeros((16,), jnp.float32)
                for d in range(HD):
                    cw = jnp.full((16,), hp * HD + d, jnp.int32)
                    pair = plsc.bitcast(plsc.load_gather(kv_r, [rows, cw]),
                                        jnp.bfloat16)
                    ka, kb = plsc.unpack(pair,
                                         format=plsc.PackFormat.INTERLEAVED)
                    ce = jnp.full((16,), hp * HD + d, jnp.int32)
                    co = jnp.full((16,), (hp + 4) * HD + d, jnp.int32)
                    acc_lo = acc_lo + ka * plsc.load_gather(qa_r, [rows, ce])
                    acc_hi = acc_hi + kb * plsc.load_gather(qa_r, [rows, co])
                t_lo = jnp.exp(jnp.clip(acc_lo * SCALE + bias_lo, -60.0, 60.0))
                t_hi = jnp.exp(jnp.clip(acc_hi * SCALE + bias_hi, -60.0, 60.0))
                plsc.store_scatter(comb, [rows,
                                          jnp.full((16,), HIDDEN + hp,
                                                   jnp.int32)], t_lo)
                plsc.store_scatter(comb, [rows,
                                          jnp.full((16,), HIDDEN + hp + 4,
                                                   jnp.int32)], t_hi)
                for d in range(HD):
                    cv = jnp.full((16,), VOFF + hp * HD + d, jnp.int32)
                    pair = plsc.bitcast(plsc.load_gather(kv_r, [rows, cv]),
                                        jnp.bfloat16)
                    va, vb = plsc.unpack(pair,
                                         format=plsc.PackFormat.INTERLEAVED)
                    ce = jnp.full((16,), hp * HD + d, jnp.int32)
                    co = jnp.full((16,), (hp + 4) * HD + d, jnp.int32)
                    plsc.store_scatter(comb, [rows, ce], va * t_lo)
                    plsc.store_scatter(comb, [rows, co], vb * t_hi)
            return 0
        lax.fori_loop(0, B // 16, group, 0)

    fire_idx(0, 0)
    fire_idx(1, 1)
    wait_idx(0)
    fire_gathers(0, 0)

    def body(j, _):
        for ph in range(4):
            p = ph % 2
            b = j * 4 + ph

            if ph >= 2:
                wait_scatter(p)
            else:
                @pl.when(j > 0)
                def _():
                    wait_scatter(p)

            @pl.when(b + 2 < NB)
            def _():
                fire_idx((ph + 2) % 4, b + 2)

            @pl.when(b + 1 < NB)
            def _():
                wait_idx((ph + 1) % 4)
                fire_gathers((ph + 1) % 2, (ph + 1) % 4)

            wait_gathers(p)
            compute(p)
            pltpu.async_copy(cbs[p], acc_sh.at[tgts[ph]], ssems[p],
                             add=True)
        return 0
    lax.fori_loop(0, NB // 4, body, 0)
    wait_scatter(0)
    wait_scatter(1)

    plsc.subcore_barrier()

    def drain(i, _):
        r0 = sid * RPS + i * B
        pltpu.sync_copy(acc_sh.at[pl.ds(r0, B)], cb0)
        pltpu.sync_copy(cb0, out_hbm.at[cid, pl.ds(r0, B)])
        return 0
    lax.fori_loop(0, RPS // B, drain, 0)


_edge_call = functools.partial(
    pl.kernel,
    out_type=jax.ShapeDtypeStruct((NC, NPAD, WA), jnp.float32),
    mesh=plsc.VectorSubcoreMesh(core_axis_name="c", subcore_axis_name="s"),
    compiler_params=pltpu.CompilerParams(use_tc_tiling_on_sc=False,
                                         needs_layout_passes=False),
    scratch_types=(
        [pltpu.VMEM((B,), jnp.int32)] * 8
        + [pltpu.VMEM((B, WQ), jnp.float32)] * 2
        + [pltpu.VMEM((B, WK), jnp.int32)] * 2
        + [pltpu.VMEM((B, WA), jnp.float32)] * 2
        + [pltpu.SemaphoreType.DMA] * 8
        + [pltpu.VMEM_SHARED((NPAD, WA), jnp.float32)]
    ),
)(_edge_body)


def _post_body(acc_ref, wo, bo, o_ref):
    a = acc_ref[0] + acc_ref[1]
    num = a[:, :HIDDEN]
    den = a[:, HIDDEN:HIDDEN + HEADS]
    recip = jnp.where(den > 0, 1.0 / den, 0.0)
    i0 = lax.broadcasted_iota(jnp.int32, (HEADS, HIDDEN), 0)
    i1 = lax.broadcasted_iota(jnp.int32, (HEADS, HIDDEN), 1)
    sel = (i1 // HD == i0).astype(jnp.float32)
    den128 = jnp.dot(recip, sel, preferred_element_type=jnp.float32)
    o_ref[...] = (jnp.dot(num * den128, wo[...],
                          preferred_element_type=jnp.float32) + bo[...])


def kernel(x, edge_index, Wq, bq, Wk, bk, Wv, bv, We, be, Wea, bea, Wo, bo):
    x2d = jnp.pad(x[0], ((0, NPAD - NN), (0, 0)))
    pad = jnp.full((EPAD - NE,), NPAD - 1, jnp.int32)
    src = jnp.concatenate([edge_index[0, :, 0], pad])
    tgt = jnp.concatenate([edge_index[0, :, 1], pad])
    weap = jnp.pad(Wea, ((0, 0), (0, HD - HEADS)))

    full = lambda s: pl.BlockSpec(s, lambda i: (0,) * len(s))
    qa, kv = pl.pallas_call(
        _pre_body,
        grid=(NPAD // R,),
        in_specs=[
            pl.BlockSpec((R, HIDDEN), lambda i: (i, 0)),
            full((HIDDEN, HIDDEN)), full((1, HIDDEN)),
            full((HIDDEN, HIDDEN)), full((1, HIDDEN)),
            full((HIDDEN, HIDDEN)), full((1, HIDDEN)),
            full((2 * HIDDEN, 64)), full((64, HD)),
        ],
        out_specs=[
            pl.BlockSpec((R, WQ), lambda i: (i, 0)),
            pl.BlockSpec((R, WK), lambda i: (i, 0)),
        ],
        out_shape=[
            jax.ShapeDtypeStruct((NPAD, WQ), jnp.float32),
            jax.ShapeDtypeStruct((NPAD, WK), jnp.int32),
        ],
    )(x2d, Wq, bq.reshape(1, -1), Wk, bk.reshape(1, -1),
      Wv, bv.reshape(1, -1), We, weap)

    acc = _edge_call(qa, kv, src, tgt)

    out = pl.pallas_call(
        _post_body,
        grid=(NN // 400,),
        in_specs=[
            pl.BlockSpec((NC, 400, WA), lambda i: (0, i, 0)),
            full((HIDDEN, HIDDEN)), full((1, HIDDEN)),
        ],
        out_specs=pl.BlockSpec((400, HIDDEN), lambda i: (i, 0)),
        out_shape=jax.ShapeDtypeStruct((NN, HIDDEN), jnp.float32),
    )(acc, Wo, bo.reshape(1, -1))
    return out.reshape(1, NN, HIDDEN)


# speedup vs baseline: 1.9882x; 1.1160x over previous
import functools

import jax
import jax.numpy as jnp
from jax import lax
from jax.experimental import pallas as pl
from jax.experimental.pallas import tpu as pltpu
from jax.experimental.pallas import tpu_sc as plsc

HIDDEN = 128
HEADS = 8
HD = 16
NN = 10000
NE = 320000
NPAD = 10240
WQ = 80
WK = 144
AOFF = 64
VOFF = 80
WA = 136
SCALE = HD ** -0.5

NC, NS = 2, 16
B = 32
NB = 316
EPAD = NC * NS * NB * B
RPS = NPAD // NS

R = 512


def _rne16(f):
    bi = lax.bitcast_convert_type(f, jnp.int32)
    return ((bi + 0x7FFF + ((bi >> 16) & 1)) >> 16) & 0xFFFF


def _pre_body(x_ref, wq, bq, wk, bk, wv, bv, we, weap,
              qa_ref, kv_ref):
    x = x_ref[...]
    wc = jnp.dot(we[...], weap[...], preferred_element_type=jnp.float32)
    q = jnp.dot(x, wq[...], preferred_element_type=jnp.float32) + bq[...]
    qa_ref[:, :AOFF] = _rne16(q[:, :AOFF]) | (_rne16(q[:, AOFF:]) << 16)
    qa_ref[:, AOFF:] = jnp.zeros((x.shape[0], WQ - AOFF), jnp.int32)
    k = (jnp.dot(x, wk[...], preferred_element_type=jnp.float32) + bk[...])
    v = (jnp.dot(x, wv[...], preferred_element_type=jnp.float32) + bv[...])
    a = jnp.dot(x, wc[:HIDDEN], preferred_element_type=jnp.float32)[:, :HEADS]
    kv_ref[:, :AOFF] = _rne16(k[:, :AOFF]) | (_rne16(k[:, AOFF:]) << 16)
    kv_ref[:, AOFF:AOFF + HEADS] = lax.bitcast_convert_type(a, jnp.int32)
    kv_ref[:, AOFF + HEADS:VOFF] = jnp.zeros((x.shape[0], VOFF - AOFF - HEADS), jnp.int32)
    kv_ref[:, VOFF:] = _rne16(v[:, :AOFF]) | (_rne16(v[:, AOFF:]) << 16)


def _edge_body(qa_hbm, kv_hbm, src_hbm, tgt_hbm, out_hbm,
               s0, s1, s2, s3, t0, t1, t2, t3,
               qa0, qa1, kv0, kv1, cb0, cb1,
               ise0, ise1, ise2, ise3, gse0, gse1, sse0, sse1,
               acc_sh):
    srcs = (s0, s1, s2, s3)
    tgts = (t0, t1, t2, t3)
    qas = (qa0, qa1)
    kvs = (kv0, kv1)
    cbs = (cb0, cb1)
    isems = (ise0, ise1, ise2, ise3)
    gsems = (gse0, gse1)
    ssems = (sse0, sse1)

    cid = lax.axis_index("c")
    sid = lax.axis_index("s")
    wid = sid * NC + cid

    zv = jnp.zeros((16,), jnp.float32)

    def zrow(r, _):
        for cc in range(WA // 16):
            cb0[r, pl.ds(cc * 16, 16)] = zv
        cb0[r, pl.ds(WA - 16, 16)] = zv
        return 0
    lax.fori_loop(0, B, zrow, 0)

    def zchunk(i, _):
        pltpu.sync_copy(cb0, acc_sh.at[pl.ds(sid * RPS + i * B, B)])
        return 0
    lax.fori_loop(0, RPS // B, zchunk, 0)
    plsc.subcore_barrier()

    lanes = lax.iota(jnp.int32, 16)

    def fire_idx(q, b):
        pltpu.async_copy(src_hbm.at[pl.ds((wid * NB + b) * B, B)], srcs[q], isems[q])
        pltpu.async_copy(tgt_hbm.at[pl.ds((wid * NB + b) * B, B)], tgts[q], isems[q])

    def wait_idx(q):
        pltpu.make_async_copy(src_hbm.at[pl.ds(0, B)], srcs[q], isems[q]).wait()
        pltpu.make_async_copy(tgt_hbm.at[pl.ds(0, B)], tgts[q], isems[q]).wait()

    def fire_gathers(p, q):
        pltpu.async_copy(qa_hbm.at[tgts[q]], qas[p], gsems[p])
        pltpu.async_copy(kv_hbm.at[srcs[q]], kvs[p], gsems[p])

    def wait_gathers(p):
        pltpu.make_async_copy(qa_hbm.at[tgts[0]], qas[p], gsems[p]).wait()
        pltpu.make_async_copy(kv_hbm.at[srcs[0]], kvs[p], gsems[p]).wait()

    def wait_scatter(p):
        pltpu.make_async_copy(cbs[p], acc_sh.at[tgts[0]], ssems[p]).wait()

    def compute(p):
        qa_r, kv_r, comb = qas[p], kvs[p], cbs[p]

        def group(g, _):
            rows = lanes + g * 16
            for hp in range(HEADS // 2):
                blo = jnp.full((16,), AOFF + hp, jnp.int32)
                bhi = jnp.full((16,), AOFF + hp + 4, jnp.int32)
                bias_lo = plsc.bitcast(plsc.load_gather(kv_r, [rows, blo]),
                                       jnp.float32)
                bias_hi = plsc.bitcast(plsc.load_gather(kv_r, [rows, bhi]),
                                       jnp.float32)
                acc_lo = jnp.zeros((16,), jnp.float32)
                acc_hi = jnp.zeros((16,), jnp.float32)
                for d in range(HD):
                    cw = jnp.full((16,), hp * HD + d, jnp.int32)
                    pair = plsc.bitcast(plsc.load_gather(kv_r, [rows, cw]),
                                        jnp.bfloat16)
                    ka, kb = plsc.unpack(pair,
                                         format=plsc.PackFormat.INTERLEAVED)
                    qpair = plsc.bitcast(
                        plsc.load_gather(qa_r, [rows, cw]), jnp.bfloat16)
                    qa_, qb_ = plsc.unpack(qpair,
                                           format=plsc.PackFormat.INTERLEAVED)
                    acc_lo = acc_lo + ka * qa_
                    acc_hi = acc_hi + kb * qb_
                t_lo = jnp.exp(jnp.clip(acc_lo * SCALE + bias_lo, -60.0, 60.0))
                t_hi = jnp.exp(jnp.clip(acc_hi * SCALE + bias_hi, -60.0, 60.0))
                plsc.store_scatter(comb, [rows,
                                          jnp.full((16,), HIDDEN + hp,
                                                   jnp.int32)], t_lo)
                plsc.store_scatter(comb, [rows,
                                          jnp.full((16,), HIDDEN + hp + 4,
                                                   jnp.int32)], t_hi)
                for d in range(HD):
                    cv = jnp.full((16,), VOFF + hp * HD + d, jnp.int32)
                    pair = plsc.bitcast(plsc.load_gather(kv_r, [rows, cv]),
                                        jnp.bfloat16)
                    va, vb = plsc.unpack(pair,
                                         format=plsc.PackFormat.INTERLEAVED)
                    ce = jnp.full((16,), hp * HD + d, jnp.int32)
                    co = jnp.full((16,), (hp + 4) * HD + d, jnp.int32)
                    plsc.store_scatter(comb, [rows, ce], va * t_lo)
                    plsc.store_scatter(comb, [rows, co], vb * t_hi)
            return 0
        lax.fori_loop(0, B // 16, group, 0)

    fire_idx(0, 0)
    fire_idx(1, 1)
    wait_idx(0)
    fire_gathers(0, 0)

    def body(j, _):
        for ph in range(4):
            p = ph % 2
            b = j * 4 + ph

            if ph >= 2:
                wait_scatter(p)
            else:
                @pl.when(j > 0)
                def _():
                    wait_scatter(p)

            @pl.when(b + 2 < NB)
            def _():
                fire_idx((ph + 2) % 4, b + 2)

            @pl.when(b + 1 < NB)
            def _():
                wait_idx((ph + 1) % 4)
                fire_gathers((ph + 1) % 2, (ph + 1) % 4)

            wait_gathers(p)
            compute(p)
            pltpu.async_copy(cbs[p], acc_sh.at[tgts[ph]], ssems[p],
                             add=True)
        return 0
    lax.fori_loop(0, NB // 4, body, 0)
    wait_scatter(0)
    wait_scatter(1)

    plsc.subcore_barrier()

    def drain(i, _):
        r0 = sid * RPS + i * B
        pltpu.sync_copy(acc_sh.at[pl.ds(r0, B)], cb0)
        pltpu.sync_copy(cb0, out_hbm.at[cid, pl.ds(r0, B)])
        return 0
    lax.fori_loop(0, RPS // B, drain, 0)


_edge_call = functools.partial(
    pl.kernel,
    out_type=jax.ShapeDtypeStruct((NC, NPAD, WA), jnp.float32),
    mesh=plsc.VectorSubcoreMesh(core_axis_name="c", subcore_axis_name="s"),
    compiler_params=pltpu.CompilerParams(use_tc_tiling_on_sc=False,
                                         needs_layout_passes=False),
    scratch_types=(
        [pltpu.VMEM((B,), jnp.int32)] * 8
        + [pltpu.VMEM((B, WQ), jnp.int32)] * 2
        + [pltpu.VMEM((B, WK), jnp.int32)] * 2
        + [pltpu.VMEM((B, WA), jnp.float32)] * 2
        + [pltpu.SemaphoreType.DMA] * 8
        + [pltpu.VMEM_SHARED((NPAD, WA), jnp.float32)]
    ),
)(_edge_body)


def _post_body(acc_ref, wo, bo, o_ref):
    a = acc_ref[0] + acc_ref[1]
    num = a[:, :HIDDEN]
    den = a[:, HIDDEN:HIDDEN + HEADS]
    recip = jnp.where(den > 0, 1.0 / den, 0.0)
    i0 = lax.broadcasted_iota(jnp.int32, (HEADS, HIDDEN), 0)
    i1 = lax.broadcasted_iota(jnp.int32, (HEADS, HIDDEN), 1)
    sel = (i1 // HD == i0).astype(jnp.float32)
    den128 = jnp.dot(recip, sel, preferred_element_type=jnp.float32)
    o_ref[...] = (jnp.dot(num * den128, wo[...],
                          preferred_element_type=jnp.float32) + bo[...])


def kernel(x, edge_index, Wq, bq, Wk, bk, Wv, bv, We, be, Wea, bea, Wo, bo):
    x2d = jnp.pad(x[0], ((0, NPAD - NN), (0, 0)))
    pad = jnp.full((EPAD - NE,), NPAD - 1, jnp.int32)
    src = jnp.concatenate([edge_index[0, :, 0], pad])
    tgt = jnp.concatenate([edge_index[0, :, 1], pad])
    weap = jnp.pad(Wea, ((0, 0), (0, HD - HEADS)))

    full = lambda s: pl.BlockSpec(s, lambda i: (0,) * len(s))
    qa, kv = pl.pallas_call(
        _pre_body,
        grid=(NPAD // R,),
        in_specs=[
            pl.BlockSpec((R, HIDDEN), lambda i: (i, 0)),
            full((HIDDEN, HIDDEN)), full((1, HIDDEN)),
            full((HIDDEN, HIDDEN)), full((1, HIDDEN)),
            full((HIDDEN, HIDDEN)), full((1, HIDDEN)),
            full((2 * HIDDEN, 64)), full((64, HD)),
        ],
        out_specs=[
            pl.BlockSpec((R, WQ), lambda i: (i, 0)),
            pl.BlockSpec((R, WK), lambda i: (i, 0)),
        ],
        out_shape=[
            jax.ShapeDtypeStruct((NPAD, WQ), jnp.int32),
            jax.ShapeDtypeStruct((NPAD, WK), jnp.int32),
        ],
    )(x2d, Wq, bq.reshape(1, -1), Wk, bk.reshape(1, -1),
      Wv, bv.reshape(1, -1), We, weap)

    acc = _edge_call(qa, kv, src, tgt)

    out = pl.pallas_call(
        _post_body,
        grid=(NN // 400,),
        in_specs=[
            pl.BlockSpec((NC, 400, WA), lambda i: (0, i, 0)),
            full((HIDDEN, HIDDEN)), full((1, HIDDEN)),
        ],
        out_specs=pl.BlockSpec((400, HIDDEN), lambda i: (i, 0)),
        out_shape=jax.ShapeDtypeStruct((NN, HIDDEN), jnp.float32),
    )(acc, Wo, bo.reshape(1, -1))
    return out.reshape(1, NN, HIDDEN)
